# baseline (speedup 1.0000x reference)
import functools

import jax
import jax.numpy as jnp
from jax import lax
from jax.experimental import pallas as pl
from jax.experimental.pallas import tpu as pltpu
from jax.experimental.pallas import tpu_sc as plsc

N_NODES = 10000
N_EDGES = 320000
DIM = 128
G = 64
OUT = 64

NCORES = 2
NSUB = 16
HALF = DIM // NCORES
EDGES_PER_T = N_EDGES // NSUB
CHUNK = 80
CHUNKS = EDGES_PER_T // CHUNK
RING = 10
ROWS_PER_TILE = 624
TAIL_ROWS = N_NODES - NSUB * ROWS_PER_TILE

BN = 2000
GRID_N = N_NODES // BN


def _sc_aggregate_body(x_hbm, src_hbm, dst_hbm, out_hbm,
                       src_v, dst_v, msg_v, acc_sh, gsem, ssem):
    c = lax.axis_index("c")
    s = lax.axis_index("s")

    zrow = msg_v.at[0]

    def zstore(j, carry):
        a = 2 * j
        for b in range(HALF // 16):
            zrow[pl.ds(a, 2), pl.ds(16 * b, 16)] = jnp.zeros(
                (2, 16), jnp.bfloat16)
        return carry

    lax.fori_loop(0, CHUNK // 2, zstore, 0)
    ZB = ROWS_PER_TILE // 8
    for blk in range(8):
        pltpu.sync_copy(
            zrow.at[pl.ds(0, ZB)],
            acc_sh.at[pl.ds(s * ROWS_PER_TILE + blk * ZB, ZB)])

    @pl.when(s == NSUB - 1)
    def _():
        base = NSUB * ROWS_PER_TILE
        pltpu.sync_copy(zrow.at[pl.ds(0, TAIL_ROWS)],
                        acc_sh.at[pl.ds(base, TAIL_ROWS)])

    pltpu.sync_copy(src_hbm.at[s], src_v)
    pltpu.sync_copy(dst_hbm.at[s], dst_v)
    plsc.subcore_barrier()

    xc = x_hbm.at[c]

    def start_gather(i, b):
        pltpu.async_copy(xc.at[src_v.at[i]], msg_v.at[b], gsem)

    def wait_gather():
        pltpu.make_async_copy(xc.at[src_v.at[0]], msg_v.at[0], gsem).wait()

    def start_scatter(i, b):
        pltpu.async_copy(msg_v.at[b], acc_sh.at[dst_v.at[i]], ssem, add=True)

    def wait_scatter():
        pltpu.make_async_copy(msg_v.at[0], acc_sh.at[dst_v.at[0]],
                              ssem).wait()

    GAHEAD = RING - 5
    SDEPTH = RING - GAHEAD
    for b in range(GAHEAD):
        start_gather(b, b)

    def group(g, carry):
        for k in range(RING):
            i = g * RING + k

            @pl.when(i >= SDEPTH)
            def _():
                wait_scatter()

            @pl.when(i + GAHEAD < CHUNKS)
            def _():
                start_gather(i + GAHEAD, (k + GAHEAD) % RING)

            wait_gather()
            start_scatter(i, k)
        return carry

    lax.fori_loop(0, CHUNKS // RING, group, 0)
    for _ in range(SDEPTH):
        wait_scatter()

    plsc.subcore_barrier()
    pltpu.sync_copy(acc_sh.at[pl.ds(s * ROWS_PER_TILE, ROWS_PER_TILE)],
                    out_hbm.at[c, pl.ds(s * ROWS_PER_TILE, ROWS_PER_TILE)])

    @pl.when(s == NSUB - 1)
    def _():
        base = NSUB * ROWS_PER_TILE
        pltpu.sync_copy(acc_sh.at[pl.ds(base, TAIL_ROWS)],
                        out_hbm.at[c, pl.ds(base, TAIL_ROWS)])


@functools.cache
def _sc_aggregate_kernel():
    return pl.kernel(
        _sc_aggregate_body,
        out_type=jax.ShapeDtypeStruct((NCORES, N_NODES, HALF), jnp.bfloat16),
        mesh=plsc.VectorSubcoreMesh(core_axis_name="c", subcore_axis_name="s",
                                    num_cores=NCORES, num_subcores=NSUB),
        scratch_types=[
            pltpu.VMEM((CHUNKS, CHUNK), jnp.int32),
            pltpu.VMEM((CHUNKS, CHUNK), jnp.int32),
            pltpu.VMEM((RING, CHUNK, HALF), jnp.bfloat16),
            pltpu.VMEM_SHARED((N_NODES, HALF), jnp.bfloat16),
            pltpu.SemaphoreType.DMA,
            pltpu.SemaphoreType.DMA,
        ],
        compiler_params=pltpu.CompilerParams(use_tc_tiling_on_sc=False),
    )


def _sc_aggregate(xs, src, dst):
    return _sc_aggregate_kernel()(xs, src, dst)


def _split_dot(hs_ref, wT_ref):
    return (jnp.dot(hs_ref[0], wT_ref[:HALF, :],
                    preferred_element_type=jnp.float32)
            + jnp.dot(hs_ref[1], wT_ref[HALF:, :],
                      preferred_element_type=jnp.float32))


def _pre_body(x_ref, wroot_ref, b_ref, xs_ref, r_ref):
    x = x_ref[...]
    xs_ref[0] = x[:, :HALF].astype(jnp.bfloat16)
    xs_ref[1] = x[:, HALF:].astype(jnp.bfloat16)
    r_ref[...] = jnp.dot(x, wroot_ref[...],
                         preferred_element_type=jnp.float32) + b_ref[...]


def _pre(x, wrootT, b):
    return pl.pallas_call(
        _pre_body,
        grid=(GRID_N,),
        in_specs=[
            pl.BlockSpec((BN, DIM), lambda i: (i, 0)),
            pl.BlockSpec((DIM, DIM), lambda i: (0, 0)),
            pl.BlockSpec((1, DIM), lambda i: (0, 0)),
        ],
        out_specs=[
            pl.BlockSpec((NCORES, BN, HALF), lambda i: (0, i, 0)),
            pl.BlockSpec((BN, DIM), lambda i: (i, 0)),
        ],
        out_shape=[
            jax.ShapeDtypeStruct((NCORES, N_NODES, HALF), jnp.bfloat16),
            jax.ShapeDtypeStruct((N_NODES, DIM), jnp.float32),
        ],
    )(x, wrootT, b)


def _mid_body(p_ref, r_ref, wrel_ref, wrootn_ref, bn_ref, hs_ref, rn_ref):
    t = jnp.maximum(_split_dot(p_ref, wrel_ref) + r_ref[...], 0.0)
    hs_ref[0] = t[:, :HALF].astype(jnp.bfloat16)
    hs_ref[1] = t[:, HALF:].astype(jnp.bfloat16)
    rn_ref[...] = jnp.dot(t, wrootn_ref[...],
                          preferred_element_type=jnp.float32) + bn_ref[...]


def _mid(p, r, wrelT, wrootnT, bn):
    return pl.pallas_call(
        _mid_body,
        grid=(GRID_N,),
        in_specs=[
            pl.BlockSpec((NCORES, BN, HALF), lambda i: (0, i, 0)),
            pl.BlockSpec((BN, DIM), lambda i: (i, 0)),
            pl.BlockSpec((DIM, DIM), lambda i: (0, 0)),
            pl.BlockSpec((DIM, DIM), lambda i: (0, 0)),
            pl.BlockSpec((1, DIM), lambda i: (0, 0)),
        ],
        out_specs=[
            pl.BlockSpec((NCORES, BN, HALF), lambda i: (0, i, 0)),
            pl.BlockSpec((BN, DIM), lambda i: (i, 0)),
        ],
        out_shape=[
            jax.ShapeDtypeStruct((NCORES, N_NODES, HALF), jnp.bfloat16),
            jax.ShapeDtypeStruct((N_NODES, DIM), jnp.float32),
        ],
    )(p, r, wrelT, wrootnT, bn)


def _final_body(p_ref, r_ref, bat_ref, wrel_ref,
                waT_ref, ba_ref, wbT_ref, bb_ref,
                emb_ref, out_ref, sums_ref, counts_ref):
    i = pl.program_id(0)

    @pl.when(i == 0)
    def _():
        sums_ref[...] = jnp.zeros_like(sums_ref)
        counts_ref[...] = jnp.zeros_like(counts_ref)

    emb = _split_dot(p_ref, wrel_ref) + r_ref[...]
    emb_ref[...] = emb
    r = jnp.maximum(emb, 0.0)

    bat = bat_ref[0, 0, :]
    gids = lax.broadcasted_iota(jnp.int32, (G, BN), 0)
    oh = (bat[None, :] == gids).astype(jnp.float32)
    sums_ref[...] += jnp.dot(oh, r, preferred_element_type=jnp.float32)
    counts_ref[...] += jnp.sum(oh, axis=1)

    @pl.when(i == GRID_N - 1)
    def _():
        pooled = sums_ref[...] / jnp.maximum(counts_ref[...], 1.0)[:, None]
        t = (jnp.dot(pooled, waT_ref[...], preferred_element_type=jnp.float32)
             + ba_ref[...])
        out_ref[...] = (jnp.dot(t, wbT_ref[...],
                                preferred_element_type=jnp.float32)
                        + bb_ref[...])


def _final(p, r, batch3, wrelT, waT, ba, wbT, bb):
    return pl.pallas_call(
        _final_body,
        grid=(GRID_N,),
        in_specs=[
            pl.BlockSpec((NCORES, BN, HALF), lambda i: (0, i, 0)),
            pl.BlockSpec((BN, DIM), lambda i: (i, 0)),
            pl.BlockSpec((1, 1, BN), lambda i: (i, 0, 0)),
            pl.BlockSpec((DIM, DIM), lambda i: (0, 0)),
            pl.BlockSpec((DIM, DIM), lambda i: (0, 0)),
            pl.BlockSpec((1, DIM), lambda i: (0, 0)),
            pl.BlockSpec((DIM, OUT), lambda i: (0, 0)),
            pl.BlockSpec((1, OUT), lambda i: (0, 0)),
        ],
        out_specs=[
            pl.BlockSpec((BN, DIM), lambda i: (i, 0)),
            pl.BlockSpec((G, OUT), lambda i: (0, 0)),
        ],
        out_shape=[
            jax.ShapeDtypeStruct((N_NODES, DIM), jnp.float32),
            jax.ShapeDtypeStruct((G, OUT), jnp.float32),
        ],
        scratch_shapes=[
            pltpu.VMEM((G, DIM), jnp.float32),
            pltpu.VMEM((G,), jnp.float32),
        ],
    )(p, r, batch3, wrelT, waT, ba, wbT, bb)


@jax.jit
def kernel(x, edge_index, edge_attr, batch,
           W1_rel, b1_rel, W1_root,
           W2_rel, b2_rel, W2_root,
           W3_rel, b3_rel, W3_root,
           Wa, ba, Wb, bb):
    del edge_attr
    src = edge_index[0].reshape(NSUB, CHUNKS, CHUNK)
    dst = edge_index[1].reshape(NSUB, CHUNKS, CHUNK)
    batch3 = batch.reshape(GRID_N, 1, BN)

    xs, r1 = _pre(x, W1_root.T, b1_rel.reshape(1, DIM))
    p1 = _sc_aggregate(xs, src, dst)
    h1, r2 = _mid(p1, r1, W1_rel.T, W2_root.T, b2_rel.reshape(1, DIM))
    p2 = _sc_aggregate(h1, src, dst)
    h2, r3 = _mid(p2, r2, W2_rel.T, W3_root.T, b3_rel.reshape(1, DIM))
    p3 = _sc_aggregate(h2, src, dst)
    emb, out = _final(p3, r3, batch3, W3_rel.T, Wa.T, ba.reshape(1, DIM),
                      Wb.T, bb.reshape(1, OUT))
    return emb, out


# speedup vs baseline: 1.0242x; 1.0242x over previous
import functools

import jax
import jax.numpy as jnp
from jax import lax
from jax.experimental import pallas as pl
from jax.experimental.pallas import tpu as pltpu
from jax.experimental.pallas import tpu_sc as plsc

N_NODES = 10000
N_EDGES = 320000
DIM = 128
G = 64
OUT = 64

NCORES = 2
NSUB = 16
HALF = DIM // NCORES
EDGES_PER_T = N_EDGES // NSUB
CHUNK = 80
CHUNKS = EDGES_PER_T // CHUNK
RING = 10
ROWS_PER_TILE = 624
TAIL_ROWS = N_NODES - NSUB * ROWS_PER_TILE

BN = 2000
GRID_N = N_NODES // BN


def _sc_aggregate_body(x_hbm, src_hbm, dst_hbm, out_hbm,
                       src_v, dst_v, msg_v, acc_sh, gsem, ssem):
    c = lax.axis_index("c")
    s = lax.axis_index("s")

    zrow = msg_v.at[0]

    def zstore(j, carry):
        a = 2 * j
        for b in range(HALF // 16):
            zrow[pl.ds(a, 2), pl.ds(16 * b, 16)] = jnp.zeros(
                (2, 16), jnp.bfloat16)
        return carry

    lax.fori_loop(0, CHUNK // 2, zstore, 0)
    ZB = ROWS_PER_TILE // 8
    for blk in range(8):
        pltpu.sync_copy(
            zrow.at[pl.ds(0, ZB)],
            acc_sh.at[pl.ds(s * ROWS_PER_TILE + blk * ZB, ZB)])

    @pl.when(s == NSUB - 1)
    def _():
        base = NSUB * ROWS_PER_TILE
        pltpu.sync_copy(zrow.at[pl.ds(0, TAIL_ROWS)],
                        acc_sh.at[pl.ds(base, TAIL_ROWS)])

    pltpu.sync_copy(src_hbm.at[s], src_v)
    pltpu.sync_copy(dst_hbm.at[s], dst_v)
    plsc.subcore_barrier()

    xc = x_hbm.at[c]

    def start_gather(i, b):
        pltpu.async_copy(xc.at[src_v.at[i]], msg_v.at[b], gsem)

    def wait_gather():
        pltpu.make_async_copy(xc.at[src_v.at[0]], msg_v.at[0], gsem).wait()

    def start_scatter(i, b):
        pltpu.async_copy(msg_v.at[b], acc_sh.at[dst_v.at[i]], ssem, add=True)

    def wait_scatter():
        pltpu.make_async_copy(msg_v.at[0], acc_sh.at[dst_v.at[0]],
                              ssem).wait()

    GAHEAD = RING - 2
    SDEPTH = RING - GAHEAD
    for b in range(GAHEAD):
        start_gather(b, b)

    def group(g, carry):
        for k in range(RING):
            i = g * RING + k

            @pl.when(i >= SDEPTH)
            def _():
                wait_scatter()

            @pl.when(i + GAHEAD < CHUNKS)
            def _():
                start_gather(i + GAHEAD, (k + GAHEAD) % RING)

            wait_gather()
            start_scatter(i, k)
        return carry

    lax.fori_loop(0, CHUNKS // RING, group, 0)
    for _ in range(SDEPTH):
        wait_scatter()

    plsc.subcore_barrier()
    pltpu.sync_copy(acc_sh.at[pl.ds(s * ROWS_PER_TILE, ROWS_PER_TILE)],
                    out_hbm.at[c, pl.ds(s * ROWS_PER_TILE, ROWS_PER_TILE)])

    @pl.when(s == NSUB - 1)
    def _():
        base = NSUB * ROWS_PER_TILE
        pltpu.sync_copy(acc_sh.at[pl.ds(base, TAIL_ROWS)],
                        out_hbm.at[c, pl.ds(base, TAIL_ROWS)])


@functools.cache
def _sc_aggregate_kernel():
    return pl.kernel(
        _sc_aggregate_body,
        out_type=jax.ShapeDtypeStruct((NCORES, N_NODES, HALF), jnp.bfloat16),
        mesh=plsc.VectorSubcoreMesh(core_axis_name="c", subcore_axis_name="s",
                                    num_cores=NCORES, num_subcores=NSUB),
        scratch_types=[
            pltpu.VMEM((CHUNKS, CHUNK), jnp.int32),
            pltpu.VMEM((CHUNKS, CHUNK), jnp.int32),
            pltpu.VMEM((RING, CHUNK, HALF), jnp.bfloat16),
            pltpu.VMEM_SHARED((N_NODES, HALF), jnp.bfloat16),
            pltpu.SemaphoreType.DMA,
            pltpu.SemaphoreType.DMA,
        ],
        compiler_params=pltpu.CompilerParams(use_tc_tiling_on_sc=False),
    )


def _sc_aggregate(xs, src, dst):
    return _sc_aggregate_kernel()(xs, src, dst)


def _split_dot(hs_ref, wT_ref):
    return (jnp.dot(hs_ref[0], wT_ref[:HALF, :],
                    preferred_element_type=jnp.float32)
            + jnp.dot(hs_ref[1], wT_ref[HALF:, :],
                      preferred_element_type=jnp.float32))


def _pre_body(x_ref, wroot_ref, b_ref, xs_ref, r_ref):
    x = x_ref[...]
    xs_ref[0] = x[:, :HALF].astype(jnp.bfloat16)
    xs_ref[1] = x[:, HALF:].astype(jnp.bfloat16)
    r_ref[...] = jnp.dot(x, wroot_ref[...],
                         preferred_element_type=jnp.float32) + b_ref[...]


def _pre(x, wrootT, b):
    return pl.pallas_call(
        _pre_body,
        grid=(GRID_N,),
        in_specs=[
            pl.BlockSpec((BN, DIM), lambda i: (i, 0)),
            pl.BlockSpec((DIM, DIM), lambda i: (0, 0)),
            pl.BlockSpec((1, DIM), lambda i: (0, 0)),
        ],
        out_specs=[
            pl.BlockSpec((NCORES, BN, HALF), lambda i: (0, i, 0)),
            pl.BlockSpec((BN, DIM), lambda i: (i, 0)),
        ],
        out_shape=[
            jax.ShapeDtypeStruct((NCORES, N_NODES, HALF), jnp.bfloat16),
            jax.ShapeDtypeStruct((N_NODES, DIM), jnp.float32),
        ],
    )(x, wrootT, b)


def _mid_body(p_ref, r_ref, wrel_ref, wrootn_ref, bn_ref, hs_ref, rn_ref):
    t = jnp.maximum(_split_dot(p_ref, wrel_ref) + r_ref[...], 0.0)
    hs_ref[0] = t[:, :HALF].astype(jnp.bfloat16)
    hs_ref[1] = t[:, HALF:].astype(jnp.bfloat16)
    rn_ref[...] = jnp.dot(t, wrootn_ref[...],
                          preferred_element_type=jnp.float32) + bn_ref[...]


def _mid(p, r, wrelT, wrootnT, bn):
    return pl.pallas_call(
        _mid_body,
        grid=(GRID_N,),
        in_specs=[
            pl.BlockSpec((NCORES, BN, HALF), lambda i: (0, i, 0)),
            pl.BlockSpec((BN, DIM), lambda i: (i, 0)),
            pl.BlockSpec((DIM, DIM), lambda i: (0, 0)),
            pl.BlockSpec((DIM, DIM), lambda i: (0, 0)),
            pl.BlockSpec((1, DIM), lambda i: (0, 0)),
        ],
        out_specs=[
            pl.BlockSpec((NCORES, BN, HALF), lambda i: (0, i, 0)),
            pl.BlockSpec((BN, DIM), lambda i: (i, 0)),
        ],
        out_shape=[
            jax.ShapeDtypeStruct((NCORES, N_NODES, HALF), jnp.bfloat16),
            jax.ShapeDtypeStruct((N_NODES, DIM), jnp.float32),
        ],
    )(p, r, wrelT, wrootnT, bn)


def _final_body(p_ref, r_ref, bat_ref, wrel_ref,
                waT_ref, ba_ref, wbT_ref, bb_ref,
                emb_ref, out_ref, sums_ref, counts_ref):
    i = pl.program_id(0)

    @pl.when(i == 0)
    def _():
        sums_ref[...] = jnp.zeros_like(sums_ref)
        counts_ref[...] = jnp.zeros_like(counts_ref)

    emb = _split_dot(p_ref, wrel_ref) + r_ref[...]
    emb_ref[...] = emb
    r = jnp.maximum(emb, 0.0)

    bat = bat_ref[0, 0, :]
    gids = lax.broadcasted_iota(jnp.int32, (G, BN), 0)
    oh = (bat[None, :] == gids).astype(jnp.float32)
    sums_ref[...] += jnp.dot(oh, r, preferred_element_type=jnp.float32)
    counts_ref[...] += jnp.sum(oh, axis=1)

    @pl.when(i == GRID_N - 1)
    def _():
        pooled = sums_ref[...] / jnp.maximum(counts_ref[...], 1.0)[:, None]
        t = (jnp.dot(pooled, waT_ref[...], preferred_element_type=jnp.float32)
             + ba_ref[...])
        out_ref[...] = (jnp.dot(t, wbT_ref[...],
                                preferred_element_type=jnp.float32)
                        + bb_ref[...])


def _final(p, r, batch3, wrelT, waT, ba, wbT, bb):
    return pl.pallas_call(
        _final_body,
        grid=(GRID_N,),
        in_specs=[
            pl.BlockSpec((NCORES, BN, HALF), lambda i: (0, i, 0)),
            pl.BlockSpec((BN, DIM), lambda i: (i, 0)),
            pl.BlockSpec((1, 1, BN), lambda i: (i, 0, 0)),
            pl.BlockSpec((DIM, DIM), lambda i: (0, 0)),
            pl.BlockSpec((DIM, DIM), lambda i: (0, 0)),
            pl.BlockSpec((1, DIM), lambda i: (0, 0)),
            pl.BlockSpec((DIM, OUT), lambda i: (0, 0)),
            pl.BlockSpec((1, OUT), lambda i: (0, 0)),
        ],
        out_specs=[
            pl.BlockSpec((BN, DIM), lambda i: (i, 0)),
            pl.BlockSpec((G, OUT), lambda i: (0, 0)),
        ],
        out_shape=[
            jax.ShapeDtypeStruct((N_NODES, DIM), jnp.float32),
            jax.ShapeDtypeStruct((G, OUT), jnp.float32),
        ],
        scratch_shapes=[
            pltpu.VMEM((G, DIM), jnp.float32),
            pltpu.VMEM((G,), jnp.float32),
        ],
    )(p, r, batch3, wrelT, waT, ba, wbT, bb)


@jax.jit
def kernel(x, edge_index, edge_attr, batch,
           W1_rel, b1_rel, W1_root,
           W2_rel, b2_rel, W2_root,
           W3_rel, b3_rel, W3_root,
           Wa, ba, Wb, bb):
    del edge_attr
    src = edge_index[0].reshape(NSUB, CHUNKS, CHUNK)
    dst = edge_index[1].reshape(NSUB, CHUNKS, CHUNK)
    batch3 = batch.reshape(GRID_N, 1, BN)

    xs, r1 = _pre(x, W1_root.T, b1_rel.reshape(1, DIM))
    p1 = _sc_aggregate(xs, src, dst)
    h1, r2 = _mid(p1, r1, W1_rel.T, W2_root.T, b2_rel.reshape(1, DIM))
    p2 = _sc_aggregate(h1, src, dst)
    h2, r3 = _mid(p2, r2, W2_rel.T, W3_root.T, b3_rel.reshape(1, DIM))
    p3 = _sc_aggregate(h2, src, dst)
    emb, out = _final(p3, r3, batch3, W3_rel.T, Wa.T, ba.reshape(1, DIM),
                      Wb.T, bb.reshape(1, OUT))
    return emb, out
